# Initial kernel scaffold; baseline (speedup 1.0000x reference)
import jax
import jax.numpy as jnp
from jax.experimental import pallas as pl


def kernel(word_seq, all_docs, Wa, Wb, Wc):
    raise NotImplementedError("write your pallas kernel here")


# speedup vs baseline: 8.5115x; 8.5115x over previous
import functools

import jax
import jax.numpy as jnp
from jax import lax
from jax.experimental import pallas as pl
from jax.experimental.pallas import tpu as pltpu
from jax.experimental.pallas import tpu_sc as plsc

VOCAB = 100000
DIM = 64
SEQ = 200
BATCH = 1024
NDOCS = 2048
SLICE = 40

NC = 2
NS = 16
L = 16
NW = NC * NS

U_PER_W = BATCH // NW
D_PER_W = NDOCS // NW
CHUNK = 4


def _sc_body(ws_hbm, ad_hbm, wa_hbm, wb_hbm, wc_hbm,
             u_out, ea_out, ec_out,
             idx_v, rows_v, stage_v, sem):
  wid = lax.axis_index("s") * NC + lax.axis_index("c")

  def run_phase(idx_hbm, table_hbm, out_hbm, jobs):
    base_job = wid * jobs
    nchunks = jobs // CHUNK

    def chunk_body(ci, carry):
      cbase = (base_job + ci * CHUNK) * SEQ
      pltpu.sync_copy(idx_hbm.at[pl.ds(cbase, CHUNK * SEQ)], idx_v)
      copies = []
      for j in range(CHUNK):
        copies.append(pltpu.async_copy(
            table_hbm.at[idx_v.at[pl.ds(j * SEQ, 128)]],
            rows_v.at[pl.ds(j * SEQ, 128)], sem))
        copies.append(pltpu.async_copy(
            table_hbm.at[idx_v.at[pl.ds(j * SEQ + 128, SEQ - 128)]],
            rows_v.at[pl.ds(j * SEQ + 128, SEQ - 128)], sem))
      for c in copies:
        c.wait()
      for j in range(CHUNK):
        def red(i, accs, j=j):
          return tuple(accs[t] + rows_v[j * SEQ + i, pl.ds(t * L, L)]
                       for t in range(4))
        accs = lax.fori_loop(
            0, SEQ, red,
            tuple(jnp.zeros((L,), jnp.float32) for _ in range(4)),
            unroll=4)
        row = ci * CHUNK + j
        for t in range(4):
          stage_v[row, pl.ds(t * L, L)] = accs[t] * (1.0 / SEQ)
      return carry

    lax.fori_loop(0, nchunks, chunk_body, 0)
    pltpu.sync_copy(stage_v.at[pl.ds(0, jobs)],
                    out_hbm.at[pl.ds(base_job, jobs)])

  run_phase(ws_hbm, wb_hbm, u_out, U_PER_W)
  run_phase(ad_hbm, wa_hbm, ea_out, D_PER_W)
  run_phase(ad_hbm, wc_hbm, ec_out, D_PER_W)


_sc_embed = pl.kernel(
    _sc_body,
    out_type=(jax.ShapeDtypeStruct((BATCH, DIM), jnp.float32),
              jax.ShapeDtypeStruct((NDOCS, DIM), jnp.float32),
              jax.ShapeDtypeStruct((NDOCS, DIM), jnp.float32)),
    mesh=plsc.VectorSubcoreMesh(core_axis_name="c", subcore_axis_name="s",
                                num_cores=NC, num_subcores=NS),
    scratch_types=[
        pltpu.VMEM((CHUNK * SEQ,), jnp.int32),
        pltpu.VMEM((CHUNK * SEQ, DIM), jnp.float32),
        pltpu.VMEM((D_PER_W, DIM), jnp.float32),
        pltpu.SemaphoreType.DMA,
    ],
    compiler_params=pltpu.CompilerParams(use_tc_tiling_on_sc=False),
)

BU = 256


def _tc_body(u_ref, m_ref, c_ref, o_ref):
  u = u_ref[...]
  m = m_ref[...]
  logits = lax.dot_general(u, m, (((1,), (1,)), ((), ())),
                           preferred_element_type=jnp.float32,
                           precision=lax.Precision.HIGHEST)
  mx = jnp.max(logits, axis=1, keepdims=True)
  e = jnp.exp(logits - mx)
  p = e / jnp.sum(e, axis=1, keepdims=True)
  o_ref[...] = lax.dot_general(p, c_ref[...], (((1,), (0,)), ((), ())),
                               preferred_element_type=jnp.float32,
                               precision=lax.Precision.HIGHEST)


_tc_attn = pl.pallas_call(
    _tc_body,
    grid=(BATCH // BU,),
    in_specs=[pl.BlockSpec((BU, DIM), lambda i: (i, 0)),
              pl.BlockSpec((NDOCS, DIM), lambda i: (0, 0)),
              pl.BlockSpec((NDOCS, DIM), lambda i: (0, 0))],
    out_specs=pl.BlockSpec((BU, DIM), lambda i: (i, 0)),
    out_shape=jax.ShapeDtypeStruct((BATCH, DIM), jnp.float32),
)


@jax.jit
def kernel(word_seq, all_docs, Wa, Wb, Wc):
  ws_flat = word_seq.T.reshape(-1)
  ad_flat = all_docs.T.reshape(-1)
  u, ea, ec = _sc_embed(ws_flat, ad_flat, Wa, Wb, Wc)
  m = jnp.concatenate([ea[:SLICE], ec[SLICE:]], axis=0)
  return _tc_attn(u, m, ea)


# speedup vs baseline: 10.8002x; 1.2689x over previous
import functools

import jax
import jax.numpy as jnp
from jax import lax
from jax.experimental import pallas as pl
from jax.experimental.pallas import tpu as pltpu
from jax.experimental.pallas import tpu_sc as plsc

VOCAB = 100000
DIM = 64
SEQ = 200
BATCH = 1024
NDOCS = 2048
SLICE = 40

NC = 2
NS = 16
L = 16
NW = NC * NS

U_PER_W = BATCH // NW
D_PER_W = NDOCS // NW
CHUNK = 4


CW = CHUNK * SEQ


def _sc_body(ws_hbm, ad_hbm, wa_hbm, wb_hbm, wc_hbm,
             u_out, ea_out, ec_out,
             idx0, idx1, rows0, rows1, stage_v,
             isem0, isem1, gsem0, gsem1):
  wid = lax.axis_index("s") * NC + lax.axis_index("c")

  def run_phase(idx_hbm, table_hbm, out_hbm, jobs):
    base_job = wid * jobs
    nchunks = jobs // CHUNK

    def fire_idx(ci, idxb, isem):
      pltpu.async_copy(idx_hbm.at[pl.ds((base_job + ci * CHUNK) * SEQ, CW)],
                       idxb, isem)

    def wait_idx(idxb, isem):
      pltpu.make_async_copy(idx_hbm.at[pl.ds(0, CW)], idxb, isem).wait()

    def fire_gathers(idxb, rowsb, gsem):
      for j in range(CHUNK):
        pltpu.async_copy(table_hbm.at[idxb.at[pl.ds(j * SEQ, 128)]],
                         rowsb.at[pl.ds(j * SEQ, 128)], gsem)
        pltpu.async_copy(table_hbm.at[idxb.at[pl.ds(j * SEQ + 128, SEQ - 128)]],
                         rowsb.at[pl.ds(j * SEQ + 128, SEQ - 128)], gsem)

    def drain_gathers(rowsb, gsem):
      pltpu.make_async_copy(table_hbm.at[pl.ds(0, CW)], rowsb, gsem).wait()

    def reduce_chunk(rowsb, ci):
      for j in range(CHUNK):
        def red(i, accs, j=j):
          return tuple(accs[t] + rowsb[j * SEQ + i, pl.ds(t * L, L)]
                       for t in range(4))
        accs = lax.fori_loop(
            0, SEQ, red,
            tuple(jnp.zeros((L,), jnp.float32) for _ in range(4)),
            unroll=4)
        row = ci * CHUNK + j
        for t in range(4):
          stage_v[row, pl.ds(t * L, L)] = accs[t] * (1.0 / SEQ)

    fire_idx(0, idx0, isem0)
    fire_idx(1, idx1, isem1)
    wait_idx(idx0, isem0)
    fire_gathers(idx0, rows0, gsem0)

    def body(k, carry):
      c0 = 2 * k
      c1 = c0 + 1
      wait_idx(idx1, isem1)
      fire_gathers(idx1, rows1, gsem1)
      drain_gathers(rows0, gsem0)

      @pl.when(c0 + 2 < nchunks)
      def _():
        fire_idx(c0 + 2, idx0, isem0)

      reduce_chunk(rows0, c0)
      drain_gathers(rows1, gsem1)

      @pl.when(c1 + 2 < nchunks)
      def _():
        fire_idx(c1 + 2, idx1, isem1)

      @pl.when(c0 + 2 < nchunks)
      def _():
        wait_idx(idx0, isem0)
        fire_gathers(idx0, rows0, gsem0)

      reduce_chunk(rows1, c1)
      return carry

    lax.fori_loop(0, nchunks // 2, body, 0)
    pltpu.sync_copy(stage_v.at[pl.ds(0, jobs)],
                    out_hbm.at[pl.ds(base_job, jobs)])

  run_phase(ws_hbm, wb_hbm, u_out, U_PER_W)
  run_phase(ad_hbm, wa_hbm, ea_out, D_PER_W)
  run_phase(ad_hbm, wc_hbm, ec_out, D_PER_W)


_sc_embed = pl.kernel(
    _sc_body,
    out_type=(jax.ShapeDtypeStruct((BATCH, DIM), jnp.float32),
              jax.ShapeDtypeStruct((NDOCS, DIM), jnp.float32),
              jax.ShapeDtypeStruct((NDOCS, DIM), jnp.float32)),
    mesh=plsc.VectorSubcoreMesh(core_axis_name="c", subcore_axis_name="s",
                                num_cores=NC, num_subcores=NS),
    scratch_types=[
        pltpu.VMEM((CW,), jnp.int32),
        pltpu.VMEM((CW,), jnp.int32),
        pltpu.VMEM((CW, DIM), jnp.float32),
        pltpu.VMEM((CW, DIM), jnp.float32),
        pltpu.VMEM((D_PER_W, DIM), jnp.float32),
        pltpu.SemaphoreType.DMA,
        pltpu.SemaphoreType.DMA,
        pltpu.SemaphoreType.DMA,
        pltpu.SemaphoreType.DMA,
    ],
    compiler_params=pltpu.CompilerParams(use_tc_tiling_on_sc=False),
)

BU = 256


def _tc_body(u_ref, m_ref, c_ref, o_ref):
  u = u_ref[...]
  m = m_ref[...]
  logits = lax.dot_general(u, m, (((1,), (1,)), ((), ())),
                           preferred_element_type=jnp.float32,
                           precision=lax.Precision.HIGHEST)
  mx = jnp.max(logits, axis=1, keepdims=True)
  e = jnp.exp(logits - mx)
  p = e / jnp.sum(e, axis=1, keepdims=True)
  o_ref[...] = lax.dot_general(p, c_ref[...], (((1,), (0,)), ((), ())),
                               preferred_element_type=jnp.float32,
                               precision=lax.Precision.HIGHEST)


_tc_attn = pl.pallas_call(
    _tc_body,
    grid=(BATCH // BU,),
    in_specs=[pl.BlockSpec((BU, DIM), lambda i: (i, 0)),
              pl.BlockSpec((NDOCS, DIM), lambda i: (0, 0)),
              pl.BlockSpec((NDOCS, DIM), lambda i: (0, 0))],
    out_specs=pl.BlockSpec((BU, DIM), lambda i: (i, 0)),
    out_shape=jax.ShapeDtypeStruct((BATCH, DIM), jnp.float32),
)


@jax.jit
def kernel(word_seq, all_docs, Wa, Wb, Wc):
  ws_flat = word_seq.T.reshape(-1)
  ad_flat = all_docs.T.reshape(-1)
  u, ea, ec = _sc_embed(ws_flat, ad_flat, Wa, Wb, Wc)
  m = jnp.concatenate([ea[:SLICE], ec[SLICE:]], axis=0)
  return _tc_attn(u, m, ea)
